# baseline (speedup 1.0000x reference)
import functools

import jax
import jax.numpy as jnp
from jax import lax
from jax.experimental import pallas as pl
from jax.experimental.pallas import tpu as pltpu
from jax.experimental.pallas import tpu_sc as plsc

NC = 2
NS = 16
K = 125
DW = 8


def _mesh():
    return plsc.VectorSubcoreMesh(core_axis_name="c", subcore_axis_name="s")


_SC_PARAMS = pltpu.CompilerParams(use_tc_tiling_on_sc=False)


def _per_tile_rows(sid, n, body_fn):
    b = (n // NS) // 8 * 8
    last = n - b * (NS - 1)

    @pl.when(sid < NS - 1)
    def _():
        body_fn(pl.multiple_of(sid * b, 8), b)

    @pl.when(sid == NS - 1)
    def _():
        body_fn((NS - 1) * b, last)


def _deg_partials(dst3, zeros_dw, ones_dw, n):
    ch = dst3.shape[1]

    @functools.partial(
        pl.kernel,
        out_type=jax.ShapeDtypeStruct((NC, n, DW), jnp.float32),
        mesh=_mesh(),
        compiler_params=_SC_PARAMS,
        scratch_types=[
            pltpu.VMEM_SHARED((n, DW), jnp.float32),
            pltpu.VMEM((ch, K), jnp.int32),
            pltpu.VMEM((K, DW), jnp.float32),
        ],
    )
    def deg_k(dst_hbm, zero_hbm, ones_hbm, out_hbm, acc, didx, ones_v):
        cid = lax.axis_index("c")
        sid = lax.axis_index("s")
        wid = sid * NC + cid
        pltpu.sync_copy(dst_hbm.at[wid], didx)
        pltpu.sync_copy(ones_hbm, ones_v)
        _per_tile_rows(sid, n, lambda base, sz: pltpu.sync_copy(
            zero_hbm.at[pl.ds(base, sz), :], acc.at[pl.ds(base, sz), :]))
        plsc.subcore_barrier()

        @pl.loop(0, ch)
        def _(ci):
            pltpu.sync_copy(ones_v, acc.at[didx.at[ci]], add=True)

        plsc.subcore_barrier()
        _per_tile_rows(sid, n, lambda base, sz: pltpu.sync_copy(
            acc.at[pl.ds(base, sz), :], out_hbm.at[cid, pl.ds(base, sz), :]))

    return deg_k(dst3, zeros_dw, ones_dw)


def _agg_partials(src3, dst3, hs, zeros_nd, n, d):
    ch = src3.shape[1]
    cb = 16
    nb = ch // cb
    assert ch % cb == 0 and cb % 2 == 0

    @functools.partial(
        pl.kernel,
        out_type=jax.ShapeDtypeStruct((NC, n, d), jnp.float32),
        mesh=_mesh(),
        compiler_params=_SC_PARAMS,
        scratch_types=[
            pltpu.VMEM_SHARED((n, d), jnp.float32),
            pltpu.VMEM_SHARED((n, d), jnp.float32),
            pltpu.VMEM((cb, K), jnp.int32),
            pltpu.VMEM((cb, K), jnp.int32),
            pltpu.VMEM((K, d), jnp.float32),
            pltpu.VMEM((K, d), jnp.float32),
            pltpu.SemaphoreType.DMA,
            pltpu.SemaphoreType.DMA,
        ],
    )
    def agg_k(src_hbm, dst_hbm, hs_hbm, zero_hbm, out_hbm,
              acc, hs_sp, sidx, didx, rows0, rows1, sem0, sem1):
        cid = lax.axis_index("c")
        sid = lax.axis_index("s")
        wid = sid * NC + cid
        _per_tile_rows(sid, n, lambda base, sz: pltpu.sync_copy(
            zero_hbm.at[pl.ds(base, sz), :], acc.at[pl.ds(base, sz), :]))
        _per_tile_rows(sid, n, lambda base, sz: pltpu.sync_copy(
            hs_hbm.at[pl.ds(base, sz), :], hs_sp.at[pl.ds(base, sz), :]))

        plsc.subcore_barrier()

        @pl.loop(0, nb)
        def _(b):
            boff = pl.multiple_of(b * cb, 8)
            pltpu.sync_copy(src_hbm.at[wid, pl.ds(boff, cb)], sidx)
            pltpu.sync_copy(dst_hbm.at[wid, pl.ds(boff, cb)], didx)
            pltpu.async_copy(hs_sp.at[sidx.at[0]], rows0, sem0)

            @pl.loop(0, cb, step=2)
            def _(ci):
                pltpu.async_copy(hs_sp.at[sidx.at[ci + 1]], rows1, sem1)
                pltpu.make_async_copy(hs_sp.at[sidx.at[ci]], rows0, sem0).wait()
                pltpu.sync_copy(rows0, acc.at[didx.at[ci]], add=True)

                @pl.when(ci + 2 < cb)
                def _():
                    pltpu.async_copy(hs_sp.at[sidx.at[ci + 2]], rows0, sem0)

                pltpu.make_async_copy(hs_sp.at[sidx.at[ci + 1]], rows1, sem1).wait()
                pltpu.sync_copy(rows1, acc.at[didx.at[ci + 1]], add=True)

        plsc.subcore_barrier()
        _per_tile_rows(sid, n, lambda base, sz: pltpu.sync_copy(
            acc.at[pl.ds(base, sz), :], out_hbm.at[cid, pl.ds(base, sz), :]))

    return agg_k(src3, dst3, hs, zeros_nd)


def _dinv_col(deg_ref):
    deg = deg_ref[0, :, 0:1] + deg_ref[1, :, 0:1] + 1.0
    return lax.rsqrt(deg)


def _row_blocks(n):
    bm = 1000 if n % 1000 == 0 else n
    return bm, n // bm


def _tc_first(deg_p, x, w1):
    n, di = x.shape
    dh = w1.shape[1]
    bm, g = _row_blocks(n)

    def body(deg_ref, x_ref, w_ref, o_ref):
        dinv = _dinv_col(deg_ref)
        h = jnp.dot(x_ref[...], w_ref[...], preferred_element_type=jnp.float32)
        o_ref[...] = h * dinv

    return pl.pallas_call(
        body,
        grid=(g,),
        in_specs=[
            pl.BlockSpec((NC, bm, DW), lambda i: (0, i, 0)),
            pl.BlockSpec((bm, di), lambda i: (i, 0)),
            pl.BlockSpec((di, dh), lambda i: (0, 0)),
        ],
        out_specs=pl.BlockSpec((bm, dh), lambda i: (i, 0)),
        out_shape=jax.ShapeDtypeStruct((n, dh), jnp.float32),
    )(deg_p, x, w1)


def _tc_mid(deg_p, p1, h1s, b1):
    n = p1.shape[1]
    dh = p1.shape[2]
    bm, g = _row_blocks(n)

    def body(deg_ref, p_ref, hs_ref, b_ref, o_ref):
        dinv = _dinv_col(deg_ref)
        s = p_ref[0] + p_ref[1] + hs_ref[...]
        h = jnp.maximum(s * dinv + b_ref[...], 0.0)
        o_ref[...] = h * dinv

    return pl.pallas_call(
        body,
        grid=(g,),
        in_specs=[
            pl.BlockSpec((NC, bm, DW), lambda i: (0, i, 0)),
            pl.BlockSpec((NC, bm, dh), lambda i: (0, i, 0)),
            pl.BlockSpec((bm, dh), lambda i: (i, 0)),
            pl.BlockSpec((dh,), lambda i: (0,)),
        ],
        out_specs=pl.BlockSpec((bm, dh), lambda i: (i, 0)),
        out_shape=jax.ShapeDtypeStruct((n, dh), jnp.float32),
    )(deg_p, p1, h1s, b1)


def _tc_last(deg_p, p2, h2s, w2, b2):
    n = p2.shape[1]
    dh = p2.shape[2]
    do = w2.shape[1]
    bm, g = _row_blocks(n)

    def body(deg_ref, p_ref, hs_ref, w_ref, b_ref, o_ref):
        dinv = _dinv_col(deg_ref)
        agg = (p_ref[0] + p_ref[1] + hs_ref[...]) * dinv
        o_ref[...] = jnp.dot(
            agg, w_ref[...], preferred_element_type=jnp.float32) + b_ref[...]

    return pl.pallas_call(
        body,
        grid=(g,),
        in_specs=[
            pl.BlockSpec((NC, bm, DW), lambda i: (0, i, 0)),
            pl.BlockSpec((NC, bm, dh), lambda i: (0, i, 0)),
            pl.BlockSpec((bm, dh), lambda i: (i, 0)),
            pl.BlockSpec((dh, do), lambda i: (0, 0)),
            pl.BlockSpec((do,), lambda i: (0,)),
        ],
        out_specs=pl.BlockSpec((bm, do), lambda i: (i, 0)),
        out_shape=jax.ShapeDtypeStruct((n, do), jnp.float32),
    )(deg_p, p2, h2s, w2, b2)


def kernel(x, edge_index, W1, b1, W2, b2):
    n = x.shape[0]
    dh = W1.shape[1]
    do = W2.shape[1]
    e = edge_index.shape[1]

    nw = NC * NS
    ch = e // (nw * K)
    src3 = edge_index[0].reshape(nw, ch, K)
    dst3 = edge_index[1].reshape(nw, ch, K)

    zeros_dw = jnp.zeros((n, DW), jnp.float32)
    ones_dw = jnp.ones((K, DW), jnp.float32)
    zeros_h = jnp.zeros((n, dh), jnp.float32)

    deg_p = _deg_partials(dst3, zeros_dw, ones_dw, n)
    h1s = _tc_first(deg_p, x, W1)
    p1 = _agg_partials(src3, dst3, h1s, zeros_h, n, dh)
    h2s = _tc_mid(deg_p, p1, h1s, b1)
    p2 = _agg_partials(src3, dst3, h2s, zeros_h, n, dh)
    return _tc_last(deg_p, p2, h2s, W2, b2)


# speedup vs baseline: 1.2409x; 1.2409x over previous
import functools

import jax
import jax.numpy as jnp
from jax import lax
from jax.experimental import pallas as pl
from jax.experimental.pallas import tpu as pltpu
from jax.experimental.pallas import tpu_sc as plsc

NC = 2
NS = 16
K = 125
DW = 8


def _mesh():
    return plsc.VectorSubcoreMesh(core_axis_name="c", subcore_axis_name="s")


_SC_PARAMS = pltpu.CompilerParams(use_tc_tiling_on_sc=False)


def _per_tile_rows(sid, n, body_fn):
    b = (n // NS) // 8 * 8
    last = n - b * (NS - 1)

    @pl.when(sid < NS - 1)
    def _():
        body_fn(pl.multiple_of(sid * b, 8), b)

    @pl.when(sid == NS - 1)
    def _():
        body_fn((NS - 1) * b, last)


def _deg_partials(dst3, zeros_dw, ones_dw, n):
    ch = dst3.shape[1]

    @functools.partial(
        pl.kernel,
        out_type=jax.ShapeDtypeStruct((NC, n), jnp.float32),
        mesh=_mesh(),
        compiler_params=_SC_PARAMS,
        scratch_types=[
            pltpu.VMEM_SHARED((n,), jnp.float32),
            pltpu.VMEM((ch, K), jnp.int32),
            pltpu.VMEM((K,), jnp.float32),
        ],
    )
    def deg_k(dst_hbm, zero_hbm, ones_hbm, out_hbm, acc, didx, ones_v):
        cid = lax.axis_index("c")
        sid = lax.axis_index("s")
        wid = sid * NC + cid
        pltpu.sync_copy(dst_hbm.at[wid], didx)
        pltpu.sync_copy(ones_hbm, ones_v)
        _per_tile_rows(sid, n, lambda base, sz: pltpu.sync_copy(
            zero_hbm.at[pl.ds(base, sz)], acc.at[pl.ds(base, sz)]))
        plsc.subcore_barrier()

        @pl.loop(0, ch)
        def _(ci):
            pltpu.sync_copy(ones_v, acc.at[didx.at[ci]], add=True)

        plsc.subcore_barrier()
        _per_tile_rows(sid, n, lambda base, sz: pltpu.sync_copy(
            acc.at[pl.ds(base, sz)], out_hbm.at[cid, pl.ds(base, sz)]))

    return deg_k(dst3, zeros_dw, ones_dw)


def _agg_partials(src3, dst3, hs, zeros_nd, n, d):
    ch = src3.shape[1]
    cb = 16
    nb = ch // cb
    assert ch % cb == 0 and cb % 2 == 0

    @functools.partial(
        pl.kernel,
        out_type=jax.ShapeDtypeStruct((NC, n, d), jnp.float32),
        mesh=_mesh(),
        compiler_params=_SC_PARAMS,
        scratch_types=[
            pltpu.VMEM_SHARED((n, d), jnp.float32),
            pltpu.VMEM_SHARED((n, d), jnp.float32),
            pltpu.VMEM((cb, K), jnp.int32),
            pltpu.VMEM((cb, K), jnp.int32),
            pltpu.VMEM((K, d), jnp.float32),
            pltpu.VMEM((K, d), jnp.float32),
            pltpu.VMEM((K, d), jnp.float32),
            pltpu.VMEM((K, d), jnp.float32),
            pltpu.SemaphoreType.DMA,
            pltpu.SemaphoreType.DMA,
            pltpu.SemaphoreType.DMA,
            pltpu.SemaphoreType.DMA,
            pltpu.SemaphoreType.DMA,
            pltpu.SemaphoreType.DMA,
            pltpu.SemaphoreType.DMA,
            pltpu.SemaphoreType.DMA,
        ],
    )
    def agg_k(src_hbm, dst_hbm, hs_hbm, zero_hbm, out_hbm,
              acc, hs_sp, sidx, didx, r0, r1, r2, r3,
              sg0, sg1, sg2, sg3, ss0, ss1, ss2, ss3):
        cid = lax.axis_index("c")
        sid = lax.axis_index("s")
        wid = sid * NC + cid
        _per_tile_rows(sid, n, lambda base, sz: pltpu.sync_copy(
            zero_hbm.at[pl.ds(base, sz), :], acc.at[pl.ds(base, sz), :]))
        _per_tile_rows(sid, n, lambda base, sz: pltpu.sync_copy(
            hs_hbm.at[pl.ds(base, sz), :], hs_sp.at[pl.ds(base, sz), :]))

        plsc.subcore_barrier()

        rows = (r0, r1, r2, r3)
        sg = (sg0, sg1, sg2, sg3)
        ss = (ss0, ss1, ss2, ss3)

        @pl.loop(0, nb)
        def _(b):
            boff = pl.multiple_of(b * cb, 8)
            pltpu.sync_copy(src_hbm.at[wid, pl.ds(boff, cb)], sidx)
            pltpu.sync_copy(dst_hbm.at[wid, pl.ds(boff, cb)], didx)
            pltpu.async_copy(hs_sp.at[sidx.at[0]], r0, sg0)
            pltpu.async_copy(hs_sp.at[sidx.at[1]], r1, sg1)

            @pl.loop(0, cb, step=4)
            def _(cj):
                for u in range(4):
                    j = cj + u
                    ru, rn = rows[u], rows[(u + 2) % 4]
                    pltpu.make_async_copy(
                        hs_sp.at[sidx.at[j]], ru, sg[u]).wait()
                    pltpu.async_copy(ru, acc.at[didx.at[j]], ss[u], add=True)

                    @pl.when(j >= 2)
                    def _():
                        pltpu.make_async_copy(
                            rn, acc.at[didx.at[j - 2]], ss[(u + 2) % 4]).wait()

                    @pl.when(j + 2 < cb)
                    def _():
                        pltpu.async_copy(
                            hs_sp.at[sidx.at[j + 2]], rn, sg[(u + 2) % 4])

            pltpu.make_async_copy(r2, acc.at[didx.at[cb - 2]], ss2).wait()
            pltpu.make_async_copy(r3, acc.at[didx.at[cb - 1]], ss3).wait()

        plsc.subcore_barrier()
        _per_tile_rows(sid, n, lambda base, sz: pltpu.sync_copy(
            acc.at[pl.ds(base, sz), :], out_hbm.at[cid, pl.ds(base, sz), :]))

    return agg_k(src3, dst3, hs, zeros_nd)


def _dinv_col(deg_ref):
    deg = deg_ref[0, :] + deg_ref[1, :] + 1.0
    return lax.rsqrt(deg)[:, None]


def _tc_first(deg_p, x, w1):
    n = x.shape[0]
    dh = w1.shape[1]

    def body(deg_ref, x_ref, w_ref, o_ref):
        dinv = _dinv_col(deg_ref)
        h = jnp.dot(x_ref[...], w_ref[...], preferred_element_type=jnp.float32)
        o_ref[...] = h * dinv

    return pl.pallas_call(
        body, out_shape=jax.ShapeDtypeStruct((n, dh), jnp.float32)
    )(deg_p, x, w1)


def _tc_mid(deg_p, p1, h1s, b1):
    n = p1.shape[1]
    dh = p1.shape[2]

    def body(deg_ref, p_ref, hs_ref, b_ref, o_ref):
        dinv = _dinv_col(deg_ref)
        s = p_ref[0] + p_ref[1] + hs_ref[...]
        h = jnp.maximum(s * dinv + b_ref[...], 0.0)
        o_ref[...] = h * dinv

    return pl.pallas_call(
        body, out_shape=jax.ShapeDtypeStruct((n, dh), jnp.float32)
    )(deg_p, p1, h1s, b1)


def _tc_last(deg_p, p2, h2s, w2, b2):
    n = p2.shape[1]
    do = w2.shape[1]

    def body(deg_ref, p_ref, hs_ref, w_ref, b_ref, o_ref):
        dinv = _dinv_col(deg_ref)
        agg = (p_ref[0] + p_ref[1] + hs_ref[...]) * dinv
        o_ref[...] = jnp.dot(
            agg, w_ref[...], preferred_element_type=jnp.float32) + b_ref[...]

    return pl.pallas_call(
        body, out_shape=jax.ShapeDtypeStruct((n, do), jnp.float32)
    )(deg_p, p2, h2s, w2, b2)


def kernel(x, edge_index, W1, b1, W2, b2):
    n = x.shape[0]
    dh = W1.shape[1]
    do = W2.shape[1]
    e = edge_index.shape[1]

    nw = NC * NS
    ch = e // (nw * K)
    src3 = edge_index[0].reshape(nw, ch, K)
    dst3 = edge_index[1].reshape(nw, ch, K)

    zeros_dw = jnp.zeros((n,), jnp.float32)
    ones_dw = jnp.ones((K,), jnp.float32)
    zeros_h = jnp.zeros((n, dh), jnp.float32)

    deg_p = _deg_partials(dst3, zeros_dw, ones_dw, n)
    h1s = _tc_first(deg_p, x, W1)
    p1 = _agg_partials(src3, dst3, h1s, zeros_h, n, dh)
    h2s = _tc_mid(deg_p, p1, h1s, b1)
    p2 = _agg_partials(src3, dst3, h2s, zeros_h, n, dh)
    return _tc_last(deg_p, p2, h2s, W2, b2)


# speedup vs baseline: 1.2966x; 1.0449x over previous
import functools

import jax
import jax.numpy as jnp
from jax import lax
from jax.experimental import pallas as pl
from jax.experimental.pallas import tpu as pltpu
from jax.experimental.pallas import tpu_sc as plsc

NC = 2
NS = 16
K = 125
DW = 8


def _mesh():
    return plsc.VectorSubcoreMesh(core_axis_name="c", subcore_axis_name="s")


_SC_PARAMS = pltpu.CompilerParams(use_tc_tiling_on_sc=False)


def _per_tile_rows(sid, n, body_fn):
    b = (n // NS) // 8 * 8
    last = n - b * (NS - 1)

    @pl.when(sid < NS - 1)
    def _():
        body_fn(pl.multiple_of(sid * b, 8), b)

    @pl.when(sid == NS - 1)
    def _():
        body_fn((NS - 1) * b, last)


def _deg_partials(dst3, zeros_dw, ones_dw, n):
    ch = dst3.shape[1]

    @functools.partial(
        pl.kernel,
        out_type=jax.ShapeDtypeStruct((NC, n), jnp.float32),
        mesh=_mesh(),
        compiler_params=_SC_PARAMS,
        scratch_types=[
            pltpu.VMEM_SHARED((n,), jnp.float32),
            pltpu.VMEM((ch, K), jnp.int32),
            pltpu.VMEM((K,), jnp.float32),
        ],
    )
    def deg_k(dst_hbm, zero_hbm, ones_hbm, out_hbm, acc, didx, ones_v):
        cid = lax.axis_index("c")
        sid = lax.axis_index("s")
        wid = sid * NC + cid
        pltpu.sync_copy(dst_hbm.at[wid], didx)
        pltpu.sync_copy(ones_hbm, ones_v)
        _per_tile_rows(sid, n, lambda base, sz: pltpu.sync_copy(
            zero_hbm.at[pl.ds(base, sz)], acc.at[pl.ds(base, sz)]))
        plsc.subcore_barrier()

        @pl.loop(0, ch)
        def _(ci):
            pltpu.sync_copy(ones_v, acc.at[didx.at[ci]], add=True)

        plsc.subcore_barrier()
        _per_tile_rows(sid, n, lambda base, sz: pltpu.sync_copy(
            acc.at[pl.ds(base, sz)], out_hbm.at[cid, pl.ds(base, sz)]))

    return deg_k(dst3, zeros_dw, ones_dw)


def _agg_partials(src3, dst3, hs, zeros_nd, n, d):
    ch = src3.shape[1]
    cb = 40 if ch % 40 == 0 else 16
    nb = ch // cb
    assert ch % cb == 0 and cb % 4 == 0

    @functools.partial(
        pl.kernel,
        out_type=jax.ShapeDtypeStruct((NC, n, d), jnp.float32),
        mesh=_mesh(),
        compiler_params=_SC_PARAMS,
        scratch_types=[
            pltpu.VMEM_SHARED((n, d), jnp.float32),
            pltpu.VMEM_SHARED((n, d), jnp.float32),
            pltpu.VMEM((cb, K), jnp.int32),
            pltpu.VMEM((cb, K), jnp.int32),
            pltpu.VMEM((K, d), jnp.float32),
            pltpu.VMEM((K, d), jnp.float32),
            pltpu.VMEM((K, d), jnp.float32),
            pltpu.VMEM((K, d), jnp.float32),
            pltpu.SemaphoreType.DMA,
            pltpu.SemaphoreType.DMA,
            pltpu.SemaphoreType.DMA,
            pltpu.SemaphoreType.DMA,
            pltpu.SemaphoreType.DMA,
            pltpu.SemaphoreType.DMA,
            pltpu.SemaphoreType.DMA,
            pltpu.SemaphoreType.DMA,
        ],
    )
    def agg_k(src_hbm, dst_hbm, hs_hbm, zero_hbm, out_hbm,
              acc, hs_sp, sidx, didx, r0, r1, r2, r3,
              sg0, sg1, sg2, sg3, ss0, ss1, ss2, ss3):
        cid = lax.axis_index("c")
        sid = lax.axis_index("s")
        wid = sid * NC + cid
        _per_tile_rows(sid, n, lambda base, sz: pltpu.sync_copy(
            zero_hbm.at[pl.ds(base, sz), :], acc.at[pl.ds(base, sz), :]))
        _per_tile_rows(sid, n, lambda base, sz: pltpu.sync_copy(
            hs_hbm.at[pl.ds(base, sz), :], hs_sp.at[pl.ds(base, sz), :]))

        plsc.subcore_barrier()

        rows = (r0, r1, r2, r3)
        sg = (sg0, sg1, sg2, sg3)
        ss = (ss0, ss1, ss2, ss3)

        @pl.loop(0, nb)
        def _(b):
            boff = pl.multiple_of(b * cb, 8)
            pltpu.sync_copy(src_hbm.at[wid, pl.ds(boff, cb)], sidx)
            pltpu.sync_copy(dst_hbm.at[wid, pl.ds(boff, cb)], didx)
            pltpu.async_copy(hs_sp.at[sidx.at[0]], r0, sg0)
            pltpu.async_copy(hs_sp.at[sidx.at[1]], r1, sg1)

            @pl.loop(0, cb, step=4)
            def _(cj):
                for u in range(4):
                    j = cj + u
                    ru, rn = rows[u], rows[(u + 2) % 4]
                    pltpu.make_async_copy(
                        hs_sp.at[sidx.at[j]], ru, sg[u]).wait()
                    pltpu.async_copy(ru, acc.at[didx.at[j]], ss[u], add=True)

                    @pl.when(j >= 2)
                    def _():
                        pltpu.make_async_copy(
                            rn, acc.at[didx.at[j - 2]], ss[(u + 2) % 4]).wait()

                    @pl.when(j + 2 < cb)
                    def _():
                        pltpu.async_copy(
                            hs_sp.at[sidx.at[j + 2]], rn, sg[(u + 2) % 4])

            pltpu.make_async_copy(r2, acc.at[didx.at[cb - 2]], ss2).wait()
            pltpu.make_async_copy(r3, acc.at[didx.at[cb - 1]], ss3).wait()

        plsc.subcore_barrier()
        _per_tile_rows(sid, n, lambda base, sz: pltpu.sync_copy(
            acc.at[pl.ds(base, sz), :], out_hbm.at[cid, pl.ds(base, sz), :]))

    return agg_k(src3, dst3, hs, zeros_nd)


def _dinv_col(deg_ref):
    deg = deg_ref[0, :] + deg_ref[1, :] + 1.0
    return lax.rsqrt(deg)[:, None]


def _tc_first(deg_p, x, w1):
    n = x.shape[0]
    dh = w1.shape[1]

    def body(deg_ref, x_ref, w_ref, o_ref):
        dinv = _dinv_col(deg_ref)
        h = jnp.dot(x_ref[...], w_ref[...], preferred_element_type=jnp.float32)
        o_ref[...] = h * dinv

    return pl.pallas_call(
        body, out_shape=jax.ShapeDtypeStruct((n, dh), jnp.float32)
    )(deg_p, x, w1)


def _tc_mid(deg_p, p1, h1s, b1):
    n = p1.shape[1]
    dh = p1.shape[2]

    def body(deg_ref, p_ref, hs_ref, b_ref, o_ref):
        dinv = _dinv_col(deg_ref)
        s = p_ref[0] + p_ref[1] + hs_ref[...]
        h = jnp.maximum(s * dinv + b_ref[...], 0.0)
        o_ref[...] = h * dinv

    return pl.pallas_call(
        body, out_shape=jax.ShapeDtypeStruct((n, dh), jnp.float32)
    )(deg_p, p1, h1s, b1)


def _tc_last(deg_p, p2, h2s, w2, b2):
    n = p2.shape[1]
    do = w2.shape[1]

    def body(deg_ref, p_ref, hs_ref, w_ref, b_ref, o_ref):
        dinv = _dinv_col(deg_ref)
        agg = (p_ref[0] + p_ref[1] + hs_ref[...]) * dinv
        o_ref[...] = jnp.dot(
            agg, w_ref[...], preferred_element_type=jnp.float32) + b_ref[...]

    return pl.pallas_call(
        body, out_shape=jax.ShapeDtypeStruct((n, do), jnp.float32)
    )(deg_p, p2, h2s, w2, b2)


def kernel(x, edge_index, W1, b1, W2, b2):
    n = x.shape[0]
    dh = W1.shape[1]
    do = W2.shape[1]
    e = edge_index.shape[1]

    nw = NC * NS
    ch = e // (nw * K)
    src3 = edge_index[0].reshape(nw, ch, K)
    dst3 = edge_index[1].reshape(nw, ch, K)

    zeros_dw = jnp.zeros((n,), jnp.float32)
    ones_dw = jnp.ones((K,), jnp.float32)
    zeros_h = jnp.zeros((n, dh), jnp.float32)

    deg_p = _deg_partials(dst3, zeros_dw, ones_dw, n)
    h1s = _tc_first(deg_p, x, W1)
    p1 = _agg_partials(src3, dst3, h1s, zeros_h, n, dh)
    h2s = _tc_mid(deg_p, p1, h1s, b1)
    p2 = _agg_partials(src3, dst3, h2s, zeros_h, n, dh)
    return _tc_last(deg_p, p2, h2s, W2, b2)


# speedup vs baseline: 1.3207x; 1.0186x over previous
import functools

import jax
import jax.numpy as jnp
from jax import lax
from jax.experimental import pallas as pl
from jax.experimental.pallas import tpu as pltpu
from jax.experimental.pallas import tpu_sc as plsc

NC = 2
NS = 16
K = 125
DW = 8


def _mesh():
    return plsc.VectorSubcoreMesh(core_axis_name="c", subcore_axis_name="s")


_SC_PARAMS = pltpu.CompilerParams(use_tc_tiling_on_sc=False)


def _per_tile_rows(sid, n, body_fn):
    b = (n // NS) // 8 * 8
    last = n - b * (NS - 1)

    @pl.when(sid < NS - 1)
    def _():
        body_fn(pl.multiple_of(sid * b, 8), b)

    @pl.when(sid == NS - 1)
    def _():
        body_fn((NS - 1) * b, last)


def _deg_partials(dst3, zeros_dw, ones_dw, n):
    ch = dst3.shape[1]

    @functools.partial(
        pl.kernel,
        out_type=jax.ShapeDtypeStruct((NC, n), jnp.float32),
        mesh=_mesh(),
        compiler_params=_SC_PARAMS,
        scratch_types=[
            pltpu.VMEM_SHARED((n,), jnp.float32),
            pltpu.VMEM((ch, K), jnp.int32),
            pltpu.VMEM((K,), jnp.float32),
            pltpu.SemaphoreType.DMA,
        ],
    )
    def deg_k(dst_hbm, zero_hbm, ones_hbm, out_hbm, acc, didx, ones_v, sem):
        cid = lax.axis_index("c")
        sid = lax.axis_index("s")
        wid = sid * NC + cid
        pltpu.sync_copy(dst_hbm.at[wid], didx)
        pltpu.sync_copy(ones_hbm, ones_v)
        _per_tile_rows(sid, n, lambda base, sz: pltpu.sync_copy(
            zero_hbm.at[pl.ds(base, sz)], acc.at[pl.ds(base, sz)]))
        plsc.subcore_barrier()

        @pl.loop(0, ch, step=8)
        def _(cj):
            for u in range(8):
                pltpu.async_copy(ones_v, acc.at[didx.at[cj + u]], sem, add=True)
            for u in range(8):
                pltpu.make_async_copy(ones_v, acc.at[didx.at[cj + u]], sem).wait()

        plsc.subcore_barrier()
        _per_tile_rows(sid, n, lambda base, sz: pltpu.sync_copy(
            acc.at[pl.ds(base, sz)], out_hbm.at[cid, pl.ds(base, sz)]))

    return deg_k(dst3, zeros_dw, ones_dw)


def _agg_partials(src3, dst3, hs, zeros_nd, n, d):
    ch = src3.shape[1]
    cb = 40 if ch % 40 == 0 else 16
    nb = ch // cb
    assert ch % cb == 0 and cb % 4 == 0

    @functools.partial(
        pl.kernel,
        out_type=jax.ShapeDtypeStruct((NC, n, d), jnp.float32),
        mesh=_mesh(),
        compiler_params=_SC_PARAMS,
        scratch_types=[
            pltpu.VMEM_SHARED((n, d), jnp.float32),
            pltpu.VMEM_SHARED((n, d), jnp.float32),
            pltpu.VMEM((cb, K), jnp.int32),
            pltpu.VMEM((cb, K), jnp.int32),
            pltpu.VMEM((K, d), jnp.float32),
            pltpu.VMEM((K, d), jnp.float32),
            pltpu.VMEM((K, d), jnp.float32),
            pltpu.VMEM((K, d), jnp.float32),
            pltpu.SemaphoreType.DMA,
            pltpu.SemaphoreType.DMA,
            pltpu.SemaphoreType.DMA,
            pltpu.SemaphoreType.DMA,
            pltpu.SemaphoreType.DMA,
            pltpu.SemaphoreType.DMA,
            pltpu.SemaphoreType.DMA,
            pltpu.SemaphoreType.DMA,
        ],
    )
    def agg_k(src_hbm, dst_hbm, hs_hbm, zero_hbm, out_hbm,
              acc, hs_sp, sidx, didx, r0, r1, r2, r3,
              sg0, sg1, sg2, sg3, ss0, ss1, ss2, ss3):
        cid = lax.axis_index("c")
        sid = lax.axis_index("s")
        wid = sid * NC + cid
        _per_tile_rows(sid, n, lambda base, sz: pltpu.sync_copy(
            zero_hbm.at[pl.ds(base, sz), :], acc.at[pl.ds(base, sz), :]))
        _per_tile_rows(sid, n, lambda base, sz: pltpu.sync_copy(
            hs_hbm.at[pl.ds(base, sz), :], hs_sp.at[pl.ds(base, sz), :]))

        plsc.subcore_barrier()

        rows = (r0, r1, r2, r3)
        sg = (sg0, sg1, sg2, sg3)
        ss = (ss0, ss1, ss2, ss3)

        @pl.loop(0, nb)
        def _(b):
            boff = pl.multiple_of(b * cb, 8)
            pltpu.sync_copy(src_hbm.at[wid, pl.ds(boff, cb)], sidx)
            pltpu.sync_copy(dst_hbm.at[wid, pl.ds(boff, cb)], didx)
            pltpu.async_copy(hs_sp.at[sidx.at[0]], r0, sg0)
            pltpu.async_copy(hs_sp.at[sidx.at[1]], r1, sg1)

            @pl.loop(0, cb, step=4)
            def _(cj):
                for u in range(4):
                    j = cj + u
                    ru, rn = rows[u], rows[(u + 2) % 4]
                    pltpu.make_async_copy(
                        hs_sp.at[sidx.at[j]], ru, sg[u]).wait()
                    pltpu.async_copy(ru, acc.at[didx.at[j]], ss[u], add=True)

                    @pl.when(j >= 2)
                    def _():
                        pltpu.make_async_copy(
                            rn, acc.at[didx.at[j - 2]], ss[(u + 2) % 4]).wait()

                    @pl.when(j + 2 < cb)
                    def _():
                        pltpu.async_copy(
                            hs_sp.at[sidx.at[j + 2]], rn, sg[(u + 2) % 4])

            pltpu.make_async_copy(r2, acc.at[didx.at[cb - 2]], ss2).wait()
            pltpu.make_async_copy(r3, acc.at[didx.at[cb - 1]], ss3).wait()

        plsc.subcore_barrier()
        _per_tile_rows(sid, n, lambda base, sz: pltpu.sync_copy(
            acc.at[pl.ds(base, sz), :], out_hbm.at[cid, pl.ds(base, sz), :]))

    return agg_k(src3, dst3, hs, zeros_nd)


def _dinv_col(deg_ref):
    deg = deg_ref[0, :] + deg_ref[1, :] + 1.0
    return lax.rsqrt(deg)[:, None]


def _tc_first(deg_p, x, w1):
    n = x.shape[0]
    dh = w1.shape[1]

    def body(deg_ref, x_ref, w_ref, o_ref):
        dinv = _dinv_col(deg_ref)
        h = jnp.dot(x_ref[...], w_ref[...], preferred_element_type=jnp.float32)
        o_ref[...] = h * dinv

    return pl.pallas_call(
        body, out_shape=jax.ShapeDtypeStruct((n, dh), jnp.float32)
    )(deg_p, x, w1)


def _tc_mid(deg_p, p1, h1s, b1):
    n = p1.shape[1]
    dh = p1.shape[2]

    def body(deg_ref, p_ref, hs_ref, b_ref, o_ref):
        dinv = _dinv_col(deg_ref)
        s = p_ref[0] + p_ref[1] + hs_ref[...]
        h = jnp.maximum(s * dinv + b_ref[...], 0.0)
        o_ref[...] = h * dinv

    return pl.pallas_call(
        body, out_shape=jax.ShapeDtypeStruct((n, dh), jnp.float32)
    )(deg_p, p1, h1s, b1)


def _tc_last(deg_p, p2, h2s, w2, b2):
    n = p2.shape[1]
    do = w2.shape[1]

    def body(deg_ref, p_ref, hs_ref, w_ref, b_ref, o_ref):
        dinv = _dinv_col(deg_ref)
        agg = (p_ref[0] + p_ref[1] + hs_ref[...]) * dinv
        o_ref[...] = jnp.dot(
            agg, w_ref[...], preferred_element_type=jnp.float32) + b_ref[...]

    return pl.pallas_call(
        body, out_shape=jax.ShapeDtypeStruct((n, do), jnp.float32)
    )(deg_p, p2, h2s, w2, b2)


def kernel(x, edge_index, W1, b1, W2, b2):
    n = x.shape[0]
    dh = W1.shape[1]
    do = W2.shape[1]
    e = edge_index.shape[1]

    nw = NC * NS
    ch = e // (nw * K)
    src3 = edge_index[0].reshape(nw, ch, K)
    dst3 = edge_index[1].reshape(nw, ch, K)

    zeros_dw = jnp.zeros((n,), jnp.float32)
    ones_dw = jnp.ones((K,), jnp.float32)
    zeros_h = jnp.zeros((n, dh), jnp.float32)

    deg_p = _deg_partials(dst3, zeros_dw, ones_dw, n)
    h1s = _tc_first(deg_p, x, W1)
    p1 = _agg_partials(src3, dst3, h1s, zeros_h, n, dh)
    h2s = _tc_mid(deg_p, p1, h1s, b1)
    p2 = _agg_partials(src3, dst3, h2s, zeros_h, n, dh)
    return _tc_last(deg_p, p2, h2s, W2, b2)
